# baseline (speedup 1.0000x reference)
import functools

import jax
import jax.numpy as jnp
from jax import lax
from jax.experimental import pallas as pl
from jax.experimental.pallas import tpu as pltpu
from jax.experimental.pallas import tpu_sc as plsc

NUM_EXPERTS = 8
TOPK = 2
HIDDEN = 2048
INTER = 4096
NTOKENS = 4096
NROWS = NTOKENS * TOPK

BM = 256
BN = 2048
P = NROWS + NUM_EXPERTS * BM
NB = P // BM
NN = HIDDEN // BN

NC = 2
NS = 16
NW = NC * NS

GCH = 8
RPW = P // NW
NPAIR = RPW // (2 * GCH)
TPW = NTOKENS // NW
TCH = 8

_mesh = plsc.VectorSubcoreMesh(core_axis_name="c", subcore_axis_name="s")
_sc_params = pltpu.CompilerParams(use_tc_tiling_on_sc=True)


@functools.partial(
    pl.kernel,
    mesh=_mesh,
    out_type=jax.ShapeDtypeStruct((P, INTER), jnp.float32),
    scratch_types=[
        pltpu.VMEM((RPW,), jnp.int32),
        pltpu.VMEM((GCH, INTER), jnp.float32),
        pltpu.VMEM((GCH, INTER), jnp.float32),
        pltpu.SemaphoreType.DMA,
        pltpu.SemaphoreType.DMA,
        pltpu.SemaphoreType.DMA,
        pltpu.SemaphoreType.DMA,
    ],
    compiler_params=_sc_params,
)
def _gather_rows(x_hbm, gidx_hbm, out_hbm, idx_v, buf0, buf1, g0, g1, w0, w1):
    wid = lax.axis_index("s") * NC + lax.axis_index("c")
    base = wid * RPW
    pltpu.sync_copy(gidx_hbm.at[pl.ds(base, RPW)], idx_v)

    def g_start(c, buf, sem):
        pltpu.async_copy(x_hbm.at[idx_v.at[pl.ds(c * GCH, GCH)]], buf, sem)

    def g_wait(c, buf, sem):
        pltpu.make_async_copy(x_hbm.at[idx_v.at[pl.ds(c * GCH, GCH)]], buf, sem).wait()

    def w_start(c, buf, sem):
        pltpu.async_copy(buf, out_hbm.at[pl.ds(base + c * GCH, GCH)], sem)

    def w_wait(c, buf, sem):
        pltpu.make_async_copy(buf, out_hbm.at[pl.ds(base + c * GCH, GCH)], sem).wait()

    g_start(0, buf0, g0)
    g_start(1, buf1, g1)

    def pair(g, _):
        c0 = 2 * g
        c1 = c0 + 1
        g_wait(c0, buf0, g0)
        w_start(c0, buf0, w0)
        g_wait(c1, buf1, g1)
        w_start(c1, buf1, w1)

        @pl.when(g < NPAIR - 1)
        def _():
            w_wait(c0, buf0, w0)
            g_start(c0 + 2, buf0, g0)
            w_wait(c1, buf1, w1)
            g_start(c1 + 2, buf1, g1)

        return ()

    lax.fori_loop(0, NPAIR, pair, ())
    w_wait(2 * NPAIR - 2, buf0, w0)
    w_wait(2 * NPAIR - 1, buf1, w1)


def _mm_body(be_ref, x_ref, w_ref, sw_ref, o_ref):
    x = x_ref[...].astype(jnp.bfloat16)
    acc = jnp.dot(x, w_ref[0], preferred_element_type=jnp.float32)
    o_ref[...] = acc * sw_ref[0, 0, :][:, None]


def _grouped_matmul(block_expert, xs, w, sw):
    grid_spec = pltpu.PrefetchScalarGridSpec(
        num_scalar_prefetch=1,
        grid=(NN, NB),
        in_specs=[
            pl.BlockSpec((BM, INTER), lambda n, m, be: (m, 0)),
            pl.BlockSpec((1, INTER, BN), lambda n, m, be: (be[m], 0, n)),
            pl.BlockSpec((1, 1, BM), lambda n, m, be: (m, 0, 0)),
        ],
        out_specs=pl.BlockSpec((BM, BN), lambda n, m, be: (m, n)),
    )
    return pl.pallas_call(
        _mm_body,
        grid_spec=grid_spec,
        out_shape=jax.ShapeDtypeStruct((P, HIDDEN), jnp.float32),
        compiler_params=pltpu.CompilerParams(
            dimension_semantics=("arbitrary", "arbitrary"),
        ),
    )(block_expert, xs, w, sw)


@functools.partial(
    pl.kernel,
    mesh=_mesh,
    out_type=jax.ShapeDtypeStruct((NTOKENS, HIDDEN), jnp.float32),
    scratch_types=[
        pltpu.VMEM((2 * TPW,), jnp.int32),
        pltpu.VMEM((2 * TCH, HIDDEN), jnp.float32),
        pltpu.VMEM((TCH, HIDDEN), jnp.float32),
        pltpu.SemaphoreType.DMA,
    ],
    compiler_params=_sc_params,
)
def _combine(y_hbm, pidx_hbm, out_hbm, idx_v, rows_v, out_v, sem):
    wid = lax.axis_index("s") * NC + lax.axis_index("c")
    tbase = wid * TPW
    pltpu.sync_copy(pidx_hbm.at[pl.ds(tbase * 2, 2 * TPW)], idx_v)

    def chunk(c, _):
        t0 = tbase + c * TCH
        pltpu.async_copy(
            y_hbm.at[idx_v.at[pl.ds(c * 2 * TCH, 2 * TCH)]], rows_v, sem)
        pltpu.make_async_copy(
            y_hbm.at[idx_v.at[pl.ds(c * 2 * TCH, 2 * TCH)]], rows_v, sem).wait()

        def jloop(j, _):
            off = j * 16
            for i in range(TCH):
                out_v[i, pl.ds(off, 16)] = (
                    rows_v[i, pl.ds(off, 16)] + rows_v[i + TCH, pl.ds(off, 16)]
                )
            return ()

        lax.fori_loop(0, HIDDEN // 16, jloop, ())
        pltpu.sync_copy(out_v, out_hbm.at[pl.ds(t0, TCH)])
        return ()

    lax.fori_loop(0, TPW // TCH, chunk, ())


def kernel(intermediate_states, w, topk_ids, topk_weight):
    flat_ids = topk_ids.reshape(-1)
    flat_w = topk_weight.reshape(-1)

    onehot = (flat_ids[:, None] == jnp.arange(NUM_EXPERTS, dtype=jnp.int32)[None, :])
    counts = jnp.sum(onehot.astype(jnp.int32), axis=0)
    padded = ((counts + BM - 1) // BM) * BM
    ends = jnp.cumsum(padded)
    starts = ends - padded
    rank = jnp.take_along_axis(
        jnp.cumsum(onehot.astype(jnp.int32), axis=0) - 1,
        flat_ids[:, None], axis=1)[:, 0]
    pos = starts[flat_ids] + rank

    gidx = jnp.zeros((P,), jnp.int32).at[pos].set(
        jnp.arange(NROWS, dtype=jnp.int32))
    sw = jnp.zeros((P,), jnp.float32).at[pos].set(flat_w)
    block_expert = jnp.clip(
        jnp.searchsorted(ends, jnp.arange(NB, dtype=jnp.int32) * BM, side="right"),
        0, NUM_EXPERTS - 1).astype(jnp.int32)

    pidx = pos.reshape(NTOKENS // TCH, TCH, TOPK).transpose(0, 2, 1).reshape(-1)
    pidx = pidx.astype(jnp.int32)

    xs = _gather_rows(intermediate_states, gidx)

    y = _grouped_matmul(block_expert, xs, w.astype(jnp.bfloat16),
                        sw.reshape(NB, 1, BM))

    return _combine(y, pidx)


# speedup vs baseline: 1.1112x; 1.1112x over previous
import functools

import jax
import jax.numpy as jnp
from jax import lax
from jax.experimental import pallas as pl
from jax.experimental.pallas import tpu as pltpu
from jax.experimental.pallas import tpu_sc as plsc

NUM_EXPERTS = 8
TOPK = 2
HIDDEN = 2048
INTER = 4096
NTOKENS = 4096
NROWS = NTOKENS * TOPK

BM = 256
BN = 1024
P = NROWS + NUM_EXPERTS * BM
NB = P // BM
NN = HIDDEN // BN

NC = 2
NS = 16
NW = NC * NS

GCH = 8
RPW = P // NW
NPAIR = RPW // (2 * GCH)
TPW = NTOKENS // NW
TCH = 8

_mesh = plsc.VectorSubcoreMesh(core_axis_name="c", subcore_axis_name="s")
_sc_params = pltpu.CompilerParams(use_tc_tiling_on_sc=True)


@functools.partial(
    pl.kernel,
    mesh=_mesh,
    out_type=jax.ShapeDtypeStruct((P, INTER), jnp.float32),
    scratch_types=[
        pltpu.VMEM((RPW,), jnp.int32),
        pltpu.VMEM((GCH, INTER), jnp.float32),
        pltpu.VMEM((GCH, INTER), jnp.float32),
        pltpu.SemaphoreType.DMA,
        pltpu.SemaphoreType.DMA,
        pltpu.SemaphoreType.DMA,
        pltpu.SemaphoreType.DMA,
    ],
    compiler_params=_sc_params,
)
def _gather_rows(x_hbm, gidx_hbm, out_hbm, idx_v, buf0, buf1, g0, g1, w0, w1):
    wid = lax.axis_index("s") * NC + lax.axis_index("c")
    base = wid * RPW
    pltpu.sync_copy(gidx_hbm.at[pl.ds(base, RPW)], idx_v)

    def g_start(c, buf, sem):
        pltpu.async_copy(x_hbm.at[idx_v.at[pl.ds(c * GCH, GCH)]], buf, sem)

    def g_wait(c, buf, sem):
        pltpu.make_async_copy(x_hbm.at[idx_v.at[pl.ds(c * GCH, GCH)]], buf, sem).wait()

    def w_start(c, buf, sem):
        pltpu.async_copy(buf, out_hbm.at[pl.ds(base + c * GCH, GCH)], sem)

    def w_wait(c, buf, sem):
        pltpu.make_async_copy(buf, out_hbm.at[pl.ds(base + c * GCH, GCH)], sem).wait()

    g_start(0, buf0, g0)
    g_start(1, buf1, g1)

    def pair(g, _):
        c0 = 2 * g
        c1 = c0 + 1
        g_wait(c0, buf0, g0)
        w_start(c0, buf0, w0)
        g_wait(c1, buf1, g1)
        w_start(c1, buf1, w1)

        @pl.when(g < NPAIR - 1)
        def _():
            w_wait(c0, buf0, w0)
            g_start(c0 + 2, buf0, g0)
            w_wait(c1, buf1, w1)
            g_start(c1 + 2, buf1, g1)

        return ()

    lax.fori_loop(0, NPAIR, pair, ())
    w_wait(2 * NPAIR - 2, buf0, w0)
    w_wait(2 * NPAIR - 1, buf1, w1)


def _mm_body(be_ref, x_ref, w_ref, sw_ref, o_ref):
    acc = jnp.dot(x_ref[...], w_ref[0], preferred_element_type=jnp.float32)
    o_ref[...] = acc * sw_ref[0, 0, :][:, None]


def _grouped_matmul(block_expert, xs, w, sw):
    grid_spec = pltpu.PrefetchScalarGridSpec(
        num_scalar_prefetch=1,
        grid=(NN, NB),
        in_specs=[
            pl.BlockSpec((BM, INTER), lambda n, m, be: (m, 0)),
            pl.BlockSpec((1, INTER, BN), lambda n, m, be: (be[m], 0, n)),
            pl.BlockSpec((1, 1, BM), lambda n, m, be: (m, 0, 0)),
        ],
        out_specs=pl.BlockSpec((BM, BN), lambda n, m, be: (m, n)),
    )
    return pl.pallas_call(
        _mm_body,
        grid_spec=grid_spec,
        out_shape=jax.ShapeDtypeStruct((P, HIDDEN), jnp.float32),
        compiler_params=pltpu.CompilerParams(
            dimension_semantics=("arbitrary", "arbitrary"),
        ),
    )(block_expert, xs, w, sw)


@functools.partial(
    pl.kernel,
    mesh=_mesh,
    out_type=jax.ShapeDtypeStruct((NTOKENS, HIDDEN), jnp.float32),
    scratch_types=[
        pltpu.VMEM((2 * TPW,), jnp.int32),
        pltpu.VMEM((2 * TCH, HIDDEN), jnp.float32),
        pltpu.VMEM((TCH, HIDDEN), jnp.float32),
        pltpu.SemaphoreType.DMA,
    ],
    compiler_params=_sc_params,
)
def _combine(y_hbm, pidx_hbm, out_hbm, idx_v, rows_v, out_v, sem):
    wid = lax.axis_index("s") * NC + lax.axis_index("c")
    tbase = wid * TPW
    pltpu.sync_copy(pidx_hbm.at[pl.ds(tbase * 2, 2 * TPW)], idx_v)

    def chunk(c, _):
        t0 = tbase + c * TCH
        pltpu.async_copy(
            y_hbm.at[idx_v.at[pl.ds(c * 2 * TCH, 2 * TCH)]], rows_v, sem)
        pltpu.make_async_copy(
            y_hbm.at[idx_v.at[pl.ds(c * 2 * TCH, 2 * TCH)]], rows_v, sem).wait()

        def jloop(j, _):
            off = j * 16
            for i in range(TCH):
                out_v[i, pl.ds(off, 16)] = (
                    rows_v[i, pl.ds(off, 16)] + rows_v[i + TCH, pl.ds(off, 16)]
                )
            return ()

        lax.fori_loop(0, HIDDEN // 16, jloop, ())
        pltpu.sync_copy(out_v, out_hbm.at[pl.ds(t0, TCH)])
        return ()

    lax.fori_loop(0, TPW // TCH, chunk, ())


def kernel(intermediate_states, w, topk_ids, topk_weight):
    flat_ids = topk_ids.reshape(-1)
    flat_w = topk_weight.reshape(-1)

    onehot = (flat_ids[:, None] == jnp.arange(NUM_EXPERTS, dtype=jnp.int32)[None, :])
    counts = jnp.sum(onehot.astype(jnp.int32), axis=0)
    padded = ((counts + BM - 1) // BM) * BM
    ends = jnp.cumsum(padded)
    starts = ends - padded
    rank = jnp.take_along_axis(
        jnp.cumsum(onehot.astype(jnp.int32), axis=0) - 1,
        flat_ids[:, None], axis=1)[:, 0]
    pos = starts[flat_ids] + rank

    gidx = jnp.zeros((P,), jnp.int32).at[pos].set(
        jnp.arange(NROWS, dtype=jnp.int32))
    sw = jnp.zeros((P,), jnp.float32).at[pos].set(flat_w)
    block_expert = jnp.clip(
        jnp.searchsorted(ends, jnp.arange(NB, dtype=jnp.int32) * BM, side="right"),
        0, NUM_EXPERTS - 1).astype(jnp.int32)

    pidx = pos.reshape(NTOKENS // TCH, TCH, TOPK).transpose(0, 2, 1).reshape(-1)
    pidx = pidx.astype(jnp.int32)

    xs = _gather_rows(intermediate_states, gidx)

    y = _grouped_matmul(block_expert, xs, w, sw.reshape(NB, 1, BM))

    return _combine(y, pidx)
